# Initial kernel scaffold; baseline (speedup 1.0000x reference)
import functools

import jax
import jax.numpy as jnp
from jax import lax
from jax.experimental import pallas as pl
from jax.experimental.pallas import tpu as pltpu
from jax.experimental.pallas import tpu_sc as plsc

N = 10000
E = 320000
DIN = 128
DH = 128
DOUT = 64

NC = 2
NS = 16
NW = NC * NS
EPW = E // NW
CHUNK = 125
NCHUNK = EPW // CHUNK
RPS = N // NS
CL = 16

TILE = 1000
NBLK = N // TILE
_PREC = lax.Precision.HIGHEST


def _sc_agg_body(with_counts, d,
                 x_hbm, src_hbm, dst_hbm, zrow_hbm, zcnt_hbm, ones_hbm,
                 out_hbm, cnt_hbm,
                 src_v, dst_v, rows_v, ones_v, acc_sh, cnt_sh, sem):
    cid = lax.axis_index("c")
    sid = lax.axis_index("s")
    wid = sid * NC + cid

    pltpu.sync_copy(zrow_hbm, acc_sh.at[pl.ds(sid * RPS, RPS)])
    if with_counts:
        pltpu.sync_copy(zcnt_hbm, cnt_sh.at[pl.ds(sid * RPS, RPS)])
        pltpu.sync_copy(ones_hbm, ones_v)
    pltpu.sync_copy(src_hbm.at[pl.ds(wid * NCHUNK, NCHUNK)], src_v)
    pltpu.sync_copy(dst_hbm.at[pl.ds(wid * NCHUNK, NCHUNK)], dst_v)
    plsc.subcore_barrier()

    def body(j, carry):
        pltpu.async_copy(x_hbm.at[src_v.at[j]], rows_v, sem).wait()
        pltpu.sync_copy(rows_v, acc_sh.at[dst_v.at[j]], add=True)
        if with_counts:
            pltpu.sync_copy(ones_v, cnt_sh.at[dst_v.at[j]], add=True)
        return carry

    lax.fori_loop(0, NCHUNK, body, 0)
    plsc.subcore_barrier()

    row0 = cid * N + sid * RPS
    pltpu.sync_copy(acc_sh.at[pl.ds(sid * RPS, RPS)], out_hbm.at[pl.ds(row0, RPS)])
    if with_counts:
        pltpu.sync_copy(cnt_sh.at[pl.ds(sid * RPS, RPS)], cnt_hbm.at[pl.ds(row0, RPS)])


def _make_sc_agg(d, with_counts):
    mesh = plsc.VectorSubcoreMesh(core_axis_name="c", subcore_axis_name="s")
    out_type = [jax.ShapeDtypeStruct((NC * N, d), jnp.float32)]
    if with_counts:
        out_type.append(jax.ShapeDtypeStruct((NC * N, CL), jnp.float32))
    scratch = [
        pltpu.VMEM((NCHUNK, CHUNK), jnp.int32),
        pltpu.VMEM((NCHUNK, CHUNK), jnp.int32),
        pltpu.VMEM((CHUNK, d), jnp.float32),
        pltpu.VMEM((CHUNK, CL), jnp.float32),
        pltpu.VMEM_SHARED((N, d), jnp.float32),
        pltpu.VMEM_SHARED((N, CL), jnp.float32),
        pltpu.SemaphoreType.DMA,
    ]
    body = functools.partial(_sc_agg_body, with_counts, d)
    if with_counts:
        def fn(x, src, dst, zrow, zcnt, ones):
            return pl.kernel(body, out_type=out_type, scratch_types=scratch,
                             mesh=mesh)(x, src, dst, zrow, zcnt, ones)
    else:
        def body2(x, src, dst, zrow, out, *rest):
            return body(x, src, dst, zrow, zrow, zrow, out, out, *rest)
        def fn(x, src, dst, zrow, zcnt, ones):
            return pl.kernel(body2, out_type=out_type, scratch_types=scratch,
                             mesh=mesh)(x, src, dst, zrow)
    return fn


def _dense_stats_body(p0, p1, c0, c1, h_ref, w_ref, r_ref, b_ref,
                      z_ref, st_ref):
    i = pl.program_id(0)
    cnt = c0[:, 0:1] + c1[:, 0:1]
    inv = 1.0 / jnp.maximum(cnt, 1.0)
    t = (p0[...] + p1[...]) * inv
    z = (jnp.dot(t, w_ref[...], preferred_element_type=jnp.float32,
                 precision=_PREC)
         + jnp.dot(h_ref[...], r_ref[...], preferred_element_type=jnp.float32,
                   precision=_PREC)
         + b_ref[...])
    z_ref[...] = z

    @pl.when(i == 0)
    def _():
        st_ref[...] = jnp.zeros_like(st_ref)

    st_ref[0:1, :] += jnp.sum(z, axis=0, keepdims=True)
    st_ref[1:2, :] += jnp.sum(z * z, axis=0, keepdims=True)


def _bn_relu_body(z_ref, st_ref, g_ref, bt_ref, h_ref):
    m = st_ref[0:1, :] / N
    v = st_ref[1:2, :] / N - m * m
    zn = (z_ref[...] - m) * lax.rsqrt(v + 1e-5) * g_ref[...] + bt_ref[...]
    h_ref[...] = jnp.maximum(zn, 0.0)


def _bn_relu_proj_body(z_ref, st_ref, g_ref, bt_ref, w3_ref, h_ref, p_ref):
    m = st_ref[0:1, :] / N
    v = st_ref[1:2, :] / N - m * m
    zn = (z_ref[...] - m) * lax.rsqrt(v + 1e-5) * g_ref[...] + bt_ref[...]
    hh = jnp.maximum(zn, 0.0)
    h_ref[...] = hh
    p_ref[...] = jnp.dot(hh, w3_ref[...], preferred_element_type=jnp.float32,
                         precision=_PREC)


def _final_body(p0, p1, c0, c1, h_ref, r_ref, b_ref, out_ref):
    cnt = c0[:, 0:1] + c1[:, 0:1]
    inv = 1.0 / jnp.maximum(cnt, 1.0)
    t = (p0[...] + p1[...]) * inv
    z = (t + jnp.dot(h_ref[...], r_ref[...], preferred_element_type=jnp.float32,
                     precision=_PREC)
         + b_ref[...])
    zmax = jnp.max(z, axis=1, keepdims=True)
    lse = jnp.log(jnp.sum(jnp.exp(z - zmax), axis=1, keepdims=True)) + zmax
    out_ref[...] = z - lse


def _part_specs(d):
    return [pl.BlockSpec((TILE, d), lambda i: (i, 0)),
            pl.BlockSpec((TILE, d), lambda i: (i + NBLK, 0))]


def _row_spec(d):
    return pl.BlockSpec((TILE, d), lambda i: (i, 0))


def _fixed_spec(r, c):
    return pl.BlockSpec((r, c), lambda i: (0, 0))


def _dense_stats(parts, cnts, h, W, R, b):
    return pl.pallas_call(
        _dense_stats_body,
        grid=(NBLK,),
        in_specs=_part_specs(DH) + _part_specs(CL)
        + [_row_spec(DH), _fixed_spec(DH, DH), _fixed_spec(DH, DH),
           _fixed_spec(1, DH)],
        out_specs=[_row_spec(DH), _fixed_spec(8, DH)],
        out_shape=[jax.ShapeDtypeStruct((N, DH), jnp.float32),
                   jax.ShapeDtypeStruct((8, DH), jnp.float32)],
    )(parts, parts, cnts, cnts, h, W, R, b.reshape(1, DH))


def _bn_relu(z, st, g, bt):
    return pl.pallas_call(
        _bn_relu_body,
        grid=(NBLK,),
        in_specs=[_row_spec(DH), _fixed_spec(8, DH), _fixed_spec(1, DH),
                  _fixed_spec(1, DH)],
        out_specs=_row_spec(DH),
        out_shape=jax.ShapeDtypeStruct((N, DH), jnp.float32),
    )(z, st, g.reshape(1, DH), bt.reshape(1, DH))


def _bn_relu_proj(z, st, g, bt, W3):
    return pl.pallas_call(
        _bn_relu_proj_body,
        grid=(NBLK,),
        in_specs=[_row_spec(DH), _fixed_spec(8, DH), _fixed_spec(1, DH),
                  _fixed_spec(1, DH), _fixed_spec(DH, DOUT)],
        out_specs=[_row_spec(DH), _row_spec(DOUT)],
        out_shape=[jax.ShapeDtypeStruct((N, DH), jnp.float32),
                   jax.ShapeDtypeStruct((N, DOUT), jnp.float32)],
    )(z, st, g.reshape(1, DH), bt.reshape(1, DH), W3)


def _final(parts, cnts, h, R3, b3):
    return pl.pallas_call(
        _final_body,
        grid=(NBLK,),
        in_specs=_part_specs(DOUT) + _part_specs(CL)
        + [_row_spec(DH), _fixed_spec(DH, DOUT), _fixed_spec(1, DOUT)],
        out_specs=_row_spec(DOUT),
        out_shape=jax.ShapeDtypeStruct((N, DOUT), jnp.float32),
    )(parts, parts, cnts, cnts, h, R3, b3.reshape(1, DOUT))


def kernel(x, edge_index, W1, R1, b1, g1, bt1, W2, R2, b2, g2, bt2, W3, R3, b3):
    dst = edge_index[0].reshape(NW * NCHUNK, CHUNK)
    src = edge_index[1].reshape(NW * NCHUNK, CHUNK)
    zrow = jnp.zeros((RPS, DH), jnp.float32)
    zrow64 = jnp.zeros((RPS, DOUT), jnp.float32)
    zcnt = jnp.zeros((RPS, CL), jnp.float32)
    ones = jnp.ones((CHUNK, CL), jnp.float32)

    agg_cnt = _make_sc_agg(DH, True)
    agg128 = _make_sc_agg(DH, False)
    agg64 = _make_sc_agg(DOUT, False)

    s1, cnts = agg_cnt(x, src, dst, zrow, zcnt, ones)
    z1, st1 = _dense_stats(s1, cnts, x, W1, R1, b1)
    h1 = _bn_relu(z1, st1, g1, bt1)

    s2 = agg128(h1, src, dst, zrow, zcnt, ones)[0]
    z2, st2 = _dense_stats(s2, cnts, h1, W2, R2, b2)
    h2, p3 = _bn_relu_proj(z2, st2, g2, bt2, W3)

    s3 = agg64(p3, src, dst, zrow64, zcnt, ones)[0]
    return _final(s3, cnts, h2, R3, b3)


# speedup vs baseline: 7.3239x; 7.3239x over previous
import functools

import jax
import jax.numpy as jnp
from jax import lax
from jax.experimental import pallas as pl
from jax.experimental.pallas import tpu as pltpu
from jax.experimental.pallas import tpu_sc as plsc

N = 10000
E = 320000
DIN = 128
DH = 128
DOUT = 64

NC = 2
NS = 16
NW = NC * NS
EPW = E // NW
CHUNK = 125
NCHUNK = EPW // CHUNK
RSTEP = 624
RSPAN = 640

TILE = 1000
NBLK = N // TILE
_PREC = lax.Precision.HIGHEST


def _sc_agg_body(x_hbm, src_hbm, dst_hbm, zrow_hbm, out_hbm,
                 src_v, dst_v, rows_v, acc_sh, sem):
    cid = lax.axis_index("c")
    sid = lax.axis_index("s")
    wid = sid * NC + cid

    pltpu.sync_copy(zrow_hbm, acc_sh.at[pl.ds(sid * RSTEP, RSPAN)])
    pltpu.sync_copy(src_hbm.at[pl.ds(wid * NCHUNK, NCHUNK)], src_v)
    pltpu.sync_copy(dst_hbm.at[pl.ds(wid * NCHUNK, NCHUNK)], dst_v)
    plsc.subcore_barrier()

    def body(j, carry):
        pltpu.async_copy(x_hbm.at[src_v.at[j]], rows_v, sem).wait()
        pltpu.sync_copy(rows_v, acc_sh.at[dst_v.at[j]], add=True)
        return carry

    lax.fori_loop(0, NCHUNK, body, 0)
    plsc.subcore_barrier()

    row0 = cid * N + sid * RSTEP
    pltpu.sync_copy(acc_sh.at[pl.ds(sid * RSTEP, RSPAN)],
                    out_hbm.at[pl.ds(row0, RSPAN)])


def _make_sc_agg(d):
    mesh = plsc.VectorSubcoreMesh(core_axis_name="c", subcore_axis_name="s")
    out_type = jax.ShapeDtypeStruct((NC * N, d), jnp.float32)
    scratch = [
        pltpu.VMEM((NCHUNK, CHUNK), jnp.int32),
        pltpu.VMEM((NCHUNK, CHUNK), jnp.int32),
        pltpu.VMEM((CHUNK, d), jnp.float32),
        pltpu.VMEM_SHARED((N, d), jnp.float32),
        pltpu.SemaphoreType.DMA,
    ]
    return pl.kernel(_sc_agg_body, out_type=out_type, scratch_types=scratch,
                     mesh=mesh)


def _sc_cnt_body(dst_hbm, zcnt_hbm, ones_hbm, cnt_hbm,
                 dst_v, ones_v, cnt_sh):
    cid = lax.axis_index("c")
    sid = lax.axis_index("s")
    wid = sid * NC + cid

    pltpu.sync_copy(zcnt_hbm, cnt_sh.at[pl.ds(sid * RSTEP, RSPAN)])
    pltpu.sync_copy(ones_hbm, ones_v)
    pltpu.sync_copy(dst_hbm.at[pl.ds(wid * NCHUNK, NCHUNK)], dst_v)
    plsc.subcore_barrier()

    def body(j, carry):
        pltpu.sync_copy(ones_v, cnt_sh.at[dst_v.at[j]], add=True)
        return carry

    lax.fori_loop(0, NCHUNK, body, 0)
    plsc.subcore_barrier()

    row0 = cid * N + sid * RSTEP
    pltpu.sync_copy(cnt_sh.at[pl.ds(sid * RSTEP, RSPAN)],
                    cnt_hbm.at[pl.ds(row0, RSPAN)])


def _make_sc_cnt():
    mesh = plsc.VectorSubcoreMesh(core_axis_name="c", subcore_axis_name="s")
    out_type = jax.ShapeDtypeStruct((NC * N, DH), jnp.float32)
    scratch = [
        pltpu.VMEM((NCHUNK, CHUNK), jnp.int32),
        pltpu.VMEM((CHUNK, DH), jnp.float32),
        pltpu.VMEM_SHARED((N, DH), jnp.float32),
    ]
    return pl.kernel(_sc_cnt_body, out_type=out_type, scratch_types=scratch,
                     mesh=mesh)


def _dense_stats_body(p0, p1, c0, c1, h_ref, w_ref, r_ref, b_ref,
                      z_ref, st_ref):
    i = pl.program_id(0)
    cnt = c0[:, 0:1] + c1[:, 0:1]
    inv = 1.0 / jnp.maximum(cnt, 1.0)
    t = (p0[...] + p1[...]) * inv
    z = (jnp.dot(t, w_ref[...], preferred_element_type=jnp.float32,
                 precision=_PREC)
         + jnp.dot(h_ref[...], r_ref[...], preferred_element_type=jnp.float32,
                   precision=_PREC)
         + b_ref[...])
    z_ref[...] = z

    @pl.when(i == 0)
    def _():
        st_ref[...] = jnp.zeros_like(st_ref)

    st_ref[0:1, :] += jnp.sum(z, axis=0, keepdims=True)
    st_ref[1:2, :] += jnp.sum(z * z, axis=0, keepdims=True)


def _bn_relu_body(z_ref, st_ref, g_ref, bt_ref, h_ref):
    m = st_ref[0:1, :] / N
    v = st_ref[1:2, :] / N - m * m
    zn = (z_ref[...] - m) * lax.rsqrt(v + 1e-5) * g_ref[...] + bt_ref[...]
    h_ref[...] = jnp.maximum(zn, 0.0)


def _final_body(p0, p1, c0, c1, h_ref, w_ref, r_ref, b_ref, out_ref):
    cnt = c0[:, 0:1] + c1[:, 0:1]
    inv = 1.0 / jnp.maximum(cnt, 1.0)
    t = (p0[...] + p1[...]) * inv
    z = (jnp.dot(t, w_ref[...], preferred_element_type=jnp.float32,
                 precision=_PREC)
         + jnp.dot(h_ref[...], r_ref[...], preferred_element_type=jnp.float32,
                   precision=_PREC)
         + b_ref[...])
    zmax = jnp.max(z, axis=1, keepdims=True)
    lse = jnp.log(jnp.sum(jnp.exp(z - zmax), axis=1, keepdims=True)) + zmax
    out_ref[...] = z - lse


def _part_specs(d):
    return [pl.BlockSpec((TILE, d), lambda i: (i, 0)),
            pl.BlockSpec((TILE, d), lambda i: (i + NBLK, 0))]


def _row_spec(d):
    return pl.BlockSpec((TILE, d), lambda i: (i, 0))


def _fixed_spec(r, c):
    return pl.BlockSpec((r, c), lambda i: (0, 0))


def _dense_stats(parts, cnts, h, W, R, b):
    return pl.pallas_call(
        _dense_stats_body,
        grid=(NBLK,),
        in_specs=_part_specs(DH) + _part_specs(DH)
        + [_row_spec(DH), _fixed_spec(DH, DH), _fixed_spec(DH, DH),
           _fixed_spec(1, DH)],
        out_specs=[_row_spec(DH), _fixed_spec(8, DH)],
        out_shape=[jax.ShapeDtypeStruct((N, DH), jnp.float32),
                   jax.ShapeDtypeStruct((8, DH), jnp.float32)],
    )(parts, parts, cnts, cnts, h, W, R, b.reshape(1, DH))


def _bn_relu(z, st, g, bt):
    return pl.pallas_call(
        _bn_relu_body,
        grid=(NBLK,),
        in_specs=[_row_spec(DH), _fixed_spec(8, DH), _fixed_spec(1, DH),
                  _fixed_spec(1, DH)],
        out_specs=_row_spec(DH),
        out_shape=jax.ShapeDtypeStruct((N, DH), jnp.float32),
    )(z, st, g.reshape(1, DH), bt.reshape(1, DH))


def _final(parts, cnts, h, W3, R3, b3):
    return pl.pallas_call(
        _final_body,
        grid=(NBLK,),
        in_specs=_part_specs(DH) + _part_specs(DH)
        + [_row_spec(DH), _fixed_spec(DH, DOUT), _fixed_spec(DH, DOUT),
           _fixed_spec(1, DOUT)],
        out_specs=_row_spec(DOUT),
        out_shape=jax.ShapeDtypeStruct((N, DOUT), jnp.float32),
    )(parts, parts, cnts, cnts, h, W3, R3, b3.reshape(1, DOUT))


def kernel(x, edge_index, W1, R1, b1, g1, bt1, W2, R2, b2, g2, bt2, W3, R3, b3):
    dst = edge_index[0].reshape(NW * NCHUNK, CHUNK)
    src = edge_index[1].reshape(NW * NCHUNK, CHUNK)
    zrow = jnp.zeros((RSPAN, DH), jnp.float32)
    ones = jnp.ones((CHUNK, DH), jnp.float32)

    agg128 = _make_sc_agg(DH)

    cnts = _make_sc_cnt()(dst, zrow, ones)
    s1 = agg128(x, src, dst, zrow)
    z1, st1 = _dense_stats(s1, cnts, x, W1, R1, b1)
    h1 = _bn_relu(z1, st1, g1, bt1)

    s2 = agg128(h1, src, dst, zrow)
    z2, st2 = _dense_stats(s2, cnts, h1, W2, R2, b2)
    h2 = _bn_relu(z2, st2, g2, bt2)

    s3 = agg128(h2, src, dst, zrow)
    return _final(s3, cnts, h2, W3, R3, b3)
